# Initial kernel scaffold; baseline (speedup 1.0000x reference)
import functools

import jax
import jax.numpy as jnp
from jax import lax
from jax.experimental import pallas as pl
from jax.experimental.pallas import tpu as pltpu
from jax.experimental.pallas import tpu_sc as plsc

N_USER = 25000
N_NODES = 50000
EMB = 64
N_EDGES = 800000

HALF = 25000
ACC_ROWS = 25008
E_PAD = 819200
C = 512
SUB = 128
NCHUNK = E_PAD // 16 // C
ET = E_PAD // 16

_mesh = plsc.VectorSubcoreMesh(core_axis_name="c", subcore_axis_name="s")


@functools.partial(
    pl.kernel,
    out_type=jax.ShapeDtypeStruct((N_NODES, EMB), jnp.float32),
    mesh=_mesh,
    scratch_types=[
        pltpu.VMEM((C,), jnp.int32),
        pltpu.VMEM((C,), jnp.int32),
        pltpu.VMEM((C // SUB, SUB), jnp.int32),
        pltpu.VMEM((C,), jnp.float32),
        pltpu.VMEM((C, EMB), jnp.float32),
        pltpu.VMEM_SHARED((ACC_ROWS, EMB), jnp.float32),
        pltpu.SemaphoreType.DMA,
    ],
)
def _spmm(emb_hbm, col_hbm, row_hbm, val_hbm, out_hbm,
          col_v, row_v, loc_v, val_v, rows_v, acc_sh, sem):
    c = lax.axis_index("c")
    s = lax.axis_index("s")

    def _zrow(r, _):
        for q in range(EMB // 16):
            rows_v[r, pl.ds(16 * q, 16)] = jnp.zeros((16,), jnp.float32)
        return 0
    lax.fori_loop(0, C, _zrow, 0)
    zbase = s * 1563
    for z in range(3):
        pltpu.sync_copy(rows_v, acc_sh.at[pl.ds(zbase + z * C, C)])
    pltpu.sync_copy(rows_v.at[pl.ds(0, 27)], acc_sh.at[pl.ds(zbase + 3 * C, 27)])
    plsc.subcore_barrier()

    ebase = s * ET

    def _chunk(g, _):
        off = ebase + g * C
        pltpu.sync_copy(col_hbm.at[pl.ds(off, C)], col_v)
        pltpu.sync_copy(row_hbm.at[pl.ds(off, C)], row_v)
        pltpu.sync_copy(val_hbm.at[pl.ds(off, C)], val_v)
        cps = [
            pltpu.async_copy(
                emb_hbm.at[col_v.at[pl.ds(j * SUB, SUB)]],
                rows_v.at[pl.ds(j * SUB, SUB)], sem)
            for j in range(C // SUB)
        ]
        for cp in cps:
            cp.wait()
        for t in range(C // 16):
            r = row_v[pl.ds(16 * t, 16)]
            loc = r - c * HALF
            oob = (loc < 0) | (loc >= HALF)
            loc = jnp.where(oob, HALF, loc)
            j, u = divmod(t, SUB // 16)
            loc_v[j, pl.ds(16 * u, 16)] = loc
        def _scale(e, _):
            v = val_v[e]
            for q in range(EMB // 16):
                x = rows_v[e, pl.ds(16 * q, 16)]
                rows_v[e, pl.ds(16 * q, 16)] = x * v
            return 0
        lax.fori_loop(0, C, _scale, 0)
        for j in range(C // SUB):
            pltpu.sync_copy(rows_v.at[pl.ds(j * SUB, SUB)],
                            acc_sh.at[loc_v.at[j]], add=True)
        return 0

    lax.fori_loop(0, NCHUNK, _chunk, 0)
    plsc.subcore_barrier()

    pltpu.sync_copy(acc_sh.at[pl.ds(s * 1562, 1562)],
                    out_hbm.at[pl.ds(c * HALF + s * 1562, 1562)])
    @pl.when(s == 0)
    def _tail():
        pltpu.sync_copy(acc_sh.at[pl.ds(24992, 8)],
                        out_hbm.at[pl.ds(c * HALF + 24992, 8)])


def _dense_body(x_ref, w_ref, b_ref, o_ref):
    y = jnp.dot(x_ref[...], w_ref[...], preferred_element_type=jnp.float32)
    y = y + b_ref[...]
    o_ref[...] = jnp.where(y >= 0, y, 0.2 * y)


_R = 2000


def _dense(x, w, b):
    return pl.pallas_call(
        _dense_body,
        grid=(N_NODES // _R,),
        in_specs=[
            pl.BlockSpec((_R, EMB), lambda i: (i, 0)),
            pl.BlockSpec((EMB, EMB), lambda i: (0, 0)),
            pl.BlockSpec((1, EMB), lambda i: (0, 0)),
        ],
        out_specs=pl.BlockSpec((_R, EMB), lambda i: (i, 0)),
        out_shape=jax.ShapeDtypeStruct((N_NODES, EMB), jnp.float32),
    )(x, w, b)


def kernel(user_emb, item_emb, W_gc_0, b_gc_0, W_gc_1, b_gc_1, W_gc_2, b_gc_2,
           adj_val, adj_row, adj_col):
    emb = jnp.concatenate([user_emb, item_emb], axis=0)
    pad = E_PAD - N_EDGES
    col = jnp.concatenate([adj_col.astype(jnp.int32), jnp.zeros((pad,), jnp.int32)])
    row = jnp.concatenate([adj_row.astype(jnp.int32), jnp.zeros((pad,), jnp.int32)])
    val = jnp.concatenate([adj_val, jnp.zeros((pad,), jnp.float32)])
    for w, b in ((W_gc_0, b_gc_0), (W_gc_1, b_gc_1), (W_gc_2, b_gc_2)):
        agg = _spmm(emb, col, row, val)
        emb = _dense(agg, w, b)
    return emb[:N_USER, :], emb[N_USER:, :]


# speedup vs baseline: 2.4567x; 2.4567x over previous
import functools

import jax
import jax.numpy as jnp
from jax import lax
from jax.experimental import pallas as pl
from jax.experimental.pallas import tpu as pltpu
from jax.experimental.pallas import tpu_sc as plsc

N_USER = 25000
N_NODES = 50000
EMB = 64
N_EDGES = 800000

HALF = 25000
ACC_ROWS = 25088
C = 384
SUB = 128
NCHUNK = 131
ET = NCHUNK * C
E_PAD = 16 * ET

_mesh = plsc.VectorSubcoreMesh(core_axis_name="c", subcore_axis_name="s")


@functools.partial(
    pl.kernel,
    out_type=jax.ShapeDtypeStruct((N_NODES, EMB), jnp.float32),
    mesh=_mesh,
    compiler_params=pltpu.CompilerParams(use_tc_tiling_on_sc=False),
    scratch_types=[
        pltpu.VMEM((C,), jnp.int32),
        pltpu.VMEM((C,), jnp.int32),
        pltpu.VMEM((C // SUB, SUB), jnp.int32),
        pltpu.VMEM((C,), jnp.float32),
        pltpu.VMEM((C, EMB), jnp.float32),
        pltpu.VMEM_SHARED((ACC_ROWS, EMB), jnp.float32),
        pltpu.SemaphoreType.DMA,
    ],
)
def _spmm(emb_hbm, col_hbm, row_hbm, val_hbm, out_hbm,
          col_v, row_v, loc_v, val_v, rows_v, acc_sh, sem):
    c = lax.axis_index("c")
    s = lax.axis_index("s")

    def _zrow(r, _):
        for q in range(EMB // 16):
            rows_v[r, pl.ds(16 * q, 16)] = jnp.zeros((16,), jnp.float32)
        return 0
    lax.fori_loop(0, C, _zrow, 0)
    zbase = s * 1568
    for z in range(4):
        pltpu.sync_copy(rows_v, acc_sh.at[pl.ds(zbase + z * C, C)])
    pltpu.sync_copy(rows_v.at[pl.ds(0, 32)], acc_sh.at[pl.ds(zbase + 4 * C, 32)])
    plsc.subcore_barrier()

    ebase = s * ET

    def _chunk(g, _):
        off = ebase + g * C
        pltpu.sync_copy(col_hbm.at[pl.ds(off, C)], col_v)
        pltpu.sync_copy(row_hbm.at[pl.ds(off, C)], row_v)
        pltpu.sync_copy(val_hbm.at[pl.ds(off, C)], val_v)
        cps = [
            pltpu.async_copy(
                emb_hbm.at[col_v.at[pl.ds(j * SUB, SUB)]],
                rows_v.at[pl.ds(j * SUB, SUB)], sem)
            for j in range(C // SUB)
        ]
        for cp in cps:
            cp.wait()
        for t in range(C // 16):
            r = row_v[pl.ds(16 * t, 16)]
            loc = r - c * HALF
            oob = (loc < 0) | (loc >= HALF)
            loc = jnp.where(oob, HALF, loc)
            j, u = divmod(t, SUB // 16)
            loc_v[j, pl.ds(16 * u, 16)] = loc
        def _scale(bk, _):
            v16 = val_v[pl.ds(16 * bk, 16)]
            for e in range(16):
                v = v16[e]
                base = 16 * bk + e
                for q in range(EMB // 16):
                    x = rows_v[base, pl.ds(16 * q, 16)]
                    rows_v[base, pl.ds(16 * q, 16)] = x * v
            return 0
        lax.fori_loop(0, C // 16, _scale, 0)
        for j in range(C // SUB):
            pltpu.sync_copy(rows_v.at[pl.ds(j * SUB, SUB)],
                            acc_sh.at[loc_v.at[j]], add=True)
        return 0

    lax.fori_loop(0, NCHUNK, _chunk, 0)
    plsc.subcore_barrier()

    pltpu.sync_copy(acc_sh.at[pl.ds(s * 1560, 1560)],
                    out_hbm.at[pl.ds(c * HALF + s * 1560, 1560)])
    @pl.when(s == 0)
    def _tail():
        pltpu.sync_copy(acc_sh.at[pl.ds(24960, 40)],
                        out_hbm.at[pl.ds(c * HALF + 24960, 40)])


def _dense_body(x_ref, w_ref, b_ref, o_ref):
    y = jnp.dot(x_ref[...], w_ref[...], preferred_element_type=jnp.float32)
    y = y + b_ref[...]
    o_ref[...] = jnp.where(y >= 0, y, 0.2 * y)


_R = 2000


def _dense(x, w, b):
    return pl.pallas_call(
        _dense_body,
        grid=(N_NODES // _R,),
        in_specs=[
            pl.BlockSpec((_R, EMB), lambda i: (i, 0)),
            pl.BlockSpec((EMB, EMB), lambda i: (0, 0)),
            pl.BlockSpec((1, EMB), lambda i: (0, 0)),
        ],
        out_specs=pl.BlockSpec((_R, EMB), lambda i: (i, 0)),
        out_shape=jax.ShapeDtypeStruct((N_NODES, EMB), jnp.float32),
    )(x, w, b)


def kernel(user_emb, item_emb, W_gc_0, b_gc_0, W_gc_1, b_gc_1, W_gc_2, b_gc_2,
           adj_val, adj_row, adj_col):
    emb = jnp.concatenate([user_emb, item_emb], axis=0)
    pad = E_PAD - N_EDGES
    col = jnp.concatenate([adj_col.astype(jnp.int32), jnp.zeros((pad,), jnp.int32)])
    row = jnp.concatenate([adj_row.astype(jnp.int32), jnp.zeros((pad,), jnp.int32)])
    val = jnp.concatenate([adj_val, jnp.zeros((pad,), jnp.float32)])
    for w, b in ((W_gc_0, b_gc_0), (W_gc_1, b_gc_1), (W_gc_2, b_gc_2)):
        agg = _spmm(emb, col, row, val)
        emb = _dense(agg, w, b)
    return emb[:N_USER, :], emb[N_USER:, :]
